# Initial kernel scaffold; baseline (speedup 1.0000x reference)
import functools

import jax
import jax.numpy as jnp
from jax import lax
from jax.experimental import pallas as pl
from jax.experimental.pallas import tpu as pltpu
from jax.experimental.pallas import tpu_sc as plsc

N = 50000
E = 800000
HALF = 25088
NPAD = 2 * HALF
DUMP = HALF
R = HALF + 8
EP = 802816
C = 128
NEG = -3.0e38


def _tc0_body(x_ref, wl_ref, wr_ref, u_ref, v_ref):
    xb = x_ref[...]
    u_ref[...] = jnp.dot(xb, wl_ref[...], preferred_element_type=jnp.float32)
    v_ref[...] = jnp.dot(xb, wr_ref[...], preferred_element_type=jnp.float32)


def _tc0(x, wl, wr):
    BN = 2000
    dout = wl.shape[1]
    return pl.pallas_call(
        _tc0_body,
        grid=(N // BN,),
        in_specs=[pl.BlockSpec((BN, 64), lambda i: (i, 0)),
                  pl.BlockSpec((64, dout), lambda i: (0, 0)),
                  pl.BlockSpec((64, dout), lambda i: (0, 0))],
        out_specs=[pl.BlockSpec((BN, dout), lambda i: (i, 0))] * 2,
        out_shape=[jax.ShapeDtypeStruct((N, dout), jnp.float32)] * 2,
    )(x, wl, wr)


def _tcmid_body(r_ref, g_ref, b_ref, wl_ref, wr_ref, u_ref, v_ref):
    t = r_ref[...]
    mu = jnp.mean(t, axis=1, keepdims=True)
    d = t - mu
    var = jnp.mean(d * d, axis=1, keepdims=True)
    y = d * lax.rsqrt(var + 1e-5) * g_ref[...] + b_ref[...]
    h = y * jax.nn.sigmoid(y)
    u_ref[...] = jnp.dot(h, wl_ref[...], preferred_element_type=jnp.float32)
    v_ref[...] = jnp.dot(h, wr_ref[...], preferred_element_type=jnp.float32)


def _tcmid(raw, g, b, wl, wr):
    BN = 2000
    dout = wl.shape[1]
    return pl.pallas_call(
        _tcmid_body,
        grid=(N // BN,),
        in_specs=[pl.BlockSpec((BN, 64), lambda i: (i, 0)),
                  pl.BlockSpec((1, 64), lambda i: (0, 0)),
                  pl.BlockSpec((1, 64), lambda i: (0, 0)),
                  pl.BlockSpec((64, dout), lambda i: (0, 0)),
                  pl.BlockSpec((64, dout), lambda i: (0, 0))],
        out_specs=[pl.BlockSpec((BN, dout), lambda i: (i, 0))] * 2,
        out_shape=[jax.ShapeDtypeStruct((N, dout), jnp.float32)] * 2,
    )(raw, g, b, wl, wr)


def _make_scA(D):
    mesh = plsc.VectorSubcoreMesh(core_axis_name="c", subcore_axis_name="s")
    K = D // 16
    TPA = EP // 32
    NCH = TPA // C

    @functools.partial(
        pl.kernel, mesh=mesh,
        out_type=[jax.ShapeDtypeStruct((EP,), jnp.float32),
                  jax.ShapeDtypeStruct((32, 16), jnp.float32)],
        scratch_types=[
            pltpu.VMEM((D,), jnp.float32),
            pltpu.VMEM((C,), jnp.int32),
            pltpu.VMEM((C,), jnp.int32),
            pltpu.VMEM((C,), jnp.int32),
            pltpu.VMEM((C, D), jnp.float32),
            pltpu.VMEM((C, D), jnp.float32),
            pltpu.VMEM((C,), jnp.float32),
            pltpu.VMEM((16,), jnp.float32),
            pltpu.SemaphoreType.DMA,
            pltpu.SemaphoreType.DMA,
        ])
    def scA(u_hbm, v_hbm, src_hbm, dst_hbm, att_hbm, e_out, tmax_out,
            attv, sidx, didx, dcl, urows, vrows, ebuf, mbuf, sem1, sem2):
        c = lax.axis_index("c")
        s = lax.axis_index("s")
        wid = s * 2 + c
        base = wid * TPA
        pltpu.sync_copy(att_hbm, attv)

        def chunk(t, mvec):
            off = base + t * C
            pltpu.sync_copy(src_hbm.at[pl.ds(off, C)], sidx)
            pltpu.sync_copy(dst_hbm.at[pl.ds(off, C)], didx)
            for g in range(C // 16):
                dv = didx[pl.ds(g * 16, 16)]
                dcl[pl.ds(g * 16, 16)] = jnp.minimum(dv, N - 1)
            cp1 = pltpu.async_copy(u_hbm.at[sidx], urows, sem1)
            cp2 = pltpu.async_copy(v_hbm.at[dcl], vrows, sem2)
            cp1.wait()
            cp2.wait()

            def edge(j, carry):
                acc = jnp.zeros((16,), jnp.float32)
                for k in range(K):
                    tv = urows[j, pl.ds(k * 16, 16)] + vrows[j, pl.ds(k * 16, 16)]
                    lv = jnp.maximum(tv, 0.2 * tv)
                    acc = acc + lv * attv[pl.ds(k * 16, 16)]
                ebuf[j] = jnp.sum(acc)
                return carry
            lax.fori_loop(0, C, edge, 0)
            for g in range(C // 16):
                mvec = jnp.maximum(mvec, ebuf[pl.ds(g * 16, 16)])
            pltpu.sync_copy(ebuf, e_out.at[pl.ds(off, C)])
            return mvec

        mvec = lax.fori_loop(0, NCH, chunk, jnp.full((16,), NEG, jnp.float32))
        mbuf[...] = mvec
        pltpu.sync_copy(mbuf, tmax_out.at[wid])

    return scA


def _make_scB(D):
    mesh = plsc.VectorSubcoreMesh(core_axis_name="c", subcore_axis_name="s")
    K = D // 16
    TPB = EP // 16
    NCH = TPB // C
    RPT = HALF // 16
    ZC = 112
    NZ = RPT // ZC

    @functools.partial(
        pl.kernel, mesh=mesh,
        out_type=jax.ShapeDtypeStruct((NPAD, D), jnp.float32),
        scratch_types=[
            pltpu.VMEM((32, 16), jnp.float32),
            pltpu.VMEM((D,), jnp.float32),
            pltpu.VMEM((C,), jnp.int32),
            pltpu.VMEM((C,), jnp.int32),
            pltpu.VMEM((C,), jnp.int32),
            pltpu.VMEM((C, D), jnp.float32),
            pltpu.VMEM((C, D), jnp.float32),
            pltpu.VMEM((C,), jnp.float32),
            pltpu.VMEM_SHARED((R, D), jnp.float32),
            pltpu.VMEM_SHARED((R,), jnp.float32),
            pltpu.SemaphoreType.DMA,
        ])
    def scB(u_hbm, e_hbm, src_hbm, dst_hbm, tmax_hbm, bias_hbm, raw_out,
            tbuf, biasv, sidx, didx, idxb, urows, wbuf, exb, acc_sh, den_sh,
            sem1):
        c = lax.axis_index("c")
        s = lax.axis_index("s")
        pltpu.sync_copy(tmax_hbm, tbuf)
        pltpu.sync_copy(bias_hbm, biasv)
        gv = tbuf[0]
        for rr in range(1, 32):
            gv = jnp.maximum(gv, tbuf[rr])
        gmax = jnp.max(gv)

        def zrow(j, carry):
            for k in range(K):
                wbuf[j, pl.ds(k * 16, 16)] = jnp.zeros((16,), jnp.float32)
            return carry
        lax.fori_loop(0, C, zrow, 0)
        for g in range(C // 16):
            exb[pl.ds(g * 16, 16)] = jnp.zeros((16,), jnp.float32)
        rbase = s * RPT
        for q in range(NZ):
            pltpu.sync_copy(wbuf.at[pl.ds(0, ZC)],
                            acc_sh.at[pl.ds(rbase + q * ZC, ZC)])
            pltpu.sync_copy(exb.at[pl.ds(0, ZC)],
                            den_sh.at[pl.ds(rbase + q * ZC, ZC)])

        @pl.when(s == 0)
        def _():
            pltpu.sync_copy(wbuf.at[pl.ds(0, 8)], acc_sh.at[pl.ds(HALF, 8)])
            pltpu.sync_copy(exb.at[pl.ds(0, 8)], den_sh.at[pl.ds(HALF, 8)])

        plsc.subcore_barrier()

        ebase = s * TPB
        nbase = c * HALF

        def chunk(t, carry):
            off = ebase + t * C
            pltpu.sync_copy(src_hbm.at[pl.ds(off, C)], sidx)
            cp = pltpu.async_copy(u_hbm.at[sidx], urows, sem1)
            pltpu.sync_copy(dst_hbm.at[pl.ds(off, C)], didx)
            for g in range(C // 16):
                dl = didx[pl.ds(g * 16, 16)] - nbase
                ok = (dl >= 0) & (dl < HALF)
                idxb[pl.ds(g * 16, 16)] = jnp.where(ok, dl, DUMP)
            pltpu.sync_copy(e_hbm.at[pl.ds(off, C)], exb)
            for g in range(C // 16):
                ev = exb[pl.ds(g * 16, 16)]
                exb[pl.ds(g * 16, 16)] = jnp.exp(ev - gmax)
            cp.wait()

            def edge(j, cc):
                sc = exb[j]
                for k in range(K):
                    wbuf[j, pl.ds(k * 16, 16)] = urows[j, pl.ds(k * 16, 16)] * sc
                return cc
            lax.fori_loop(0, C, edge, 0)
            pltpu.sync_copy(wbuf, acc_sh.at[idxb], add=True)
            pltpu.sync_copy(exb, den_sh.at[idxb], add=True)
            return carry
        lax.fori_loop(0, NCH, chunk, 0)
        plsc.subcore_barrier()

        def fin(q, carry):
            r0 = rbase + q * ZC
            pltpu.sync_copy(acc_sh.at[pl.ds(r0, ZC)], wbuf.at[pl.ds(0, ZC)])
            pltpu.sync_copy(den_sh.at[pl.ds(r0, ZC)], exb.at[pl.ds(0, ZC)])

            def node(j, cc):
                rcp = 1.0 / (exb[j] + 1e-16)
                for k in range(K):
                    wbuf[j, pl.ds(k * 16, 16)] = (
                        wbuf[j, pl.ds(k * 16, 16)] * rcp
                        + biasv[pl.ds(k * 16, 16)])
                return cc
            lax.fori_loop(0, ZC, node, 0)
            pltpu.sync_copy(wbuf.at[pl.ds(0, ZC)],
                            raw_out.at[pl.ds(nbase + r0, ZC)])
            return carry
        lax.fori_loop(0, NZ, fin, 0)

    return scB


def kernel(x, edge_index, params):
    src = edge_index[0].astype(jnp.int32)
    dst = edge_index[1].astype(jnp.int32)
    src_p = jnp.concatenate([src, jnp.zeros((EP - E,), jnp.int32)])
    dst_p = jnp.concatenate([dst, jnp.full((EP - E,), NPAD, jnp.int32)])

    scA64, scB64 = _make_scA(64), _make_scB(64)
    scA16, scB16 = _make_scA(16), _make_scB(16)

    raw = None
    for i in range(4):
        p = params[f'layer{i}']
        wl, wr, att, b = p['Wl'], p['Wr'], p['att'], p['b']
        if i == 3:
            wl = jnp.pad(wl, ((0, 0), (0, 6)))
            wr = jnp.pad(wr, ((0, 0), (0, 6)))
            att = jnp.pad(att, (0, 6))
            b = jnp.pad(b, (0, 6))
        if i == 0:
            u, v = _tc0(x, wl, wr)
        else:
            nrm = params[f'norm{i-1}']
            u, v = _tcmid(raw[:N], nrm['g'].reshape(1, -1),
                          nrm['b'].reshape(1, -1), wl, wr)
        scA = scA64 if wl.shape[1] == 64 else scA16
        scB = scB64 if wl.shape[1] == 64 else scB16
        e, tmax = scA(u, v, src_p, dst_p, att)
        raw = scB(u, e, src_p, dst_p, tmax, b)
    return raw[:N, :10]


# speedup vs baseline: 2.4905x; 2.4905x over previous
import functools

import jax
import jax.numpy as jnp
from jax import lax
from jax.experimental import pallas as pl
from jax.experimental.pallas import tpu as pltpu
from jax.experimental.pallas import tpu_sc as plsc

N = 50000
E = 800000
OWN = 1568
NPAD = 32 * OWN
DUMPT = OWN
EP = 802816
TPA = EP // 32
C = 128
CAPSEG = TPA + 128
NSEG = 32 * 32
EPB = NSEG * CAPSEG
NEG = -3.0e38


def _tc0_body(x_ref, wl_ref, wr_ref, t_ref):
    xb = x_ref[...]
    dout = wl_ref.shape[1]
    if dout < 64:
        t_ref[...] = jnp.zeros_like(t_ref)
    t_ref[:, 0:dout] = jnp.dot(xb, wl_ref[...],
                               preferred_element_type=jnp.float32)
    t_ref[:, 64:64 + dout] = jnp.dot(xb, wr_ref[...],
                                     preferred_element_type=jnp.float32)


def _tc0(x, wl, wr):
    BN = 2000
    dout = wl.shape[1]
    return pl.pallas_call(
        _tc0_body,
        grid=(N // BN,),
        in_specs=[pl.BlockSpec((BN, 64), lambda i: (i, 0)),
                  pl.BlockSpec((64, dout), lambda i: (0, 0)),
                  pl.BlockSpec((64, dout), lambda i: (0, 0))],
        out_specs=[pl.BlockSpec((BN, 128), lambda i: (i, 0))],
        out_shape=[jax.ShapeDtypeStruct((N, 128), jnp.float32)],
    )(x, wl, wr)[0]


def _tcmid_body(r_ref, g_ref, b_ref, wl_ref, wr_ref, t_ref):
    t = r_ref[...]
    mu = jnp.mean(t, axis=1, keepdims=True)
    d = t - mu
    var = jnp.mean(d * d, axis=1, keepdims=True)
    y = d * lax.rsqrt(var + 1e-5) * g_ref[...] + b_ref[...]
    h = y * jax.nn.sigmoid(y)
    dout = wl_ref.shape[1]
    if dout < 64:
        t_ref[...] = jnp.zeros_like(t_ref)
    t_ref[:, 0:dout] = jnp.dot(h, wl_ref[...],
                               preferred_element_type=jnp.float32)
    t_ref[:, 64:64 + dout] = jnp.dot(h, wr_ref[...],
                                     preferred_element_type=jnp.float32)


def _tcmid(raw, g, b, wl, wr):
    BN = 2000
    dout = wl.shape[1]
    return pl.pallas_call(
        _tcmid_body,
        grid=(N // BN,),
        in_specs=[pl.BlockSpec((BN, 64), lambda i: (i, 0)),
                  pl.BlockSpec((1, 64), lambda i: (0, 0)),
                  pl.BlockSpec((1, 64), lambda i: (0, 0)),
                  pl.BlockSpec((64, dout), lambda i: (0, 0)),
                  pl.BlockSpec((64, dout), lambda i: (0, 0))],
        out_specs=[pl.BlockSpec((BN, 128), lambda i: (i, 0))],
        out_shape=[jax.ShapeDtypeStruct((N, 128), jnp.float32)],
    )(raw, g, b, wl, wr)[0]


def _vsum16(x):
    lane = lax.iota(jnp.int32, 16)
    for sh in (8, 4, 2, 1):
        x = x + x.at[lane ^ sh].get(mode="promise_in_bounds")
    return x


def _vmax16(x):
    lane = lax.iota(jnp.int32, 16)
    for sh in (8, 4, 2, 1):
        x = jnp.maximum(x, x.at[lane ^ sh].get(mode="promise_in_bounds"))
    return x


def _pick32(lo, hi, w):
    lane = lax.iota(jnp.int32, 16)
    a = jnp.where(lane == w, lo, jnp.zeros_like(lo))
    b = jnp.where(lane == (w - 16), hi, jnp.zeros_like(hi))
    return _vsum16(a + b)[0]


def _make_bin():
    mesh = plsc.VectorSubcoreMesh(core_axis_name="c", subcore_axis_name="s")
    NCH = TPA // C

    @functools.partial(
        pl.kernel, mesh=mesh,
        out_type=[jax.ShapeDtypeStruct((EPB,), jnp.int32),
                  jax.ShapeDtypeStruct((EPB,), jnp.int32),
                  jax.ShapeDtypeStruct((32, 32), jnp.int32)],
        scratch_types=[
            pltpu.VMEM((C,), jnp.int32),
            pltpu.VMEM((C,), jnp.int32),
            pltpu.VMEM((9216,), jnp.int32),
            pltpu.VMEM((9216,), jnp.int32),
            pltpu.VMEM((32,), jnp.int32),
        ])
    def bink(src_hbm, dst_hbm, bsrc_out, bdst_out, lens_out,
             sidx, didx, stg_s, stg_d, lensb):
        c = lax.axis_index("c")
        s = lax.axis_index("s")
        t = s * 2 + c
        base = t * TPA
        lane = lax.iota(jnp.int32, 16)

        def chunk(i, carry):
            off = base + i * C
            pltpu.sync_copy(src_hbm.at[pl.ds(off, C)], sidx)
            pltpu.sync_copy(dst_hbm.at[pl.ds(off, C)], didx)
            clo, chi = carry[0], carry[1]
            nfls = list(carry[2:])
            for g in range(C // 16):
                sv = sidx[pl.ds(g * 16, 16)]
                dv = didx[pl.ds(g * 16, 16)]
                bv = jnp.minimum(
                    jnp.right_shift(
                        jnp.right_shift(dv, 5) * 1338, 16), 31)
                rank = jnp.zeros((16,), jnp.int32)
                for sh in range(1, 16):
                    idxc = jnp.maximum(lane - sh, 0)
                    gsh = bv.at[idxc].get(mode="promise_in_bounds")
                    rank = rank + jnp.where((lane >= sh) & (gsh == bv),
                                            1, 0)
                cl = clo.at[jnp.minimum(bv, 15)].get(
                    mode="promise_in_bounds")
                ch = chi.at[jnp.maximum(bv - 16, 0)].get(
                    mode="promise_in_bounds")
                dest = bv * 288 + jnp.where(bv < 16, cl, ch) + rank
                for j16 in range(16):
                    p = dest[j16]
                    stg_s[pl.ds(p, 16)] = (jnp.zeros((16,), jnp.int32)
                                           + sv[j16])
                    stg_d[pl.ds(p, 16)] = (jnp.zeros((16,), jnp.int32)
                                           + dv[j16])
                    bj = bv[j16]
                    clo = clo + jnp.where(lane == bj, 1, 0)
                    chi = chi + jnp.where(lane == (bj - 16), 1, 0)
            outs = []
            nclo, nchi = clo, chi
            for b in range(32):
                cur = (clo[b] if b < 16 else chi[b - 16])
                nfl = nfls[b]
                seg = (b * 32 + t) * CAPSEG + nfl * 128
                pltpu.sync_copy(stg_s.at[pl.ds(b * 288, 128)],
                                bsrc_out.at[pl.ds(seg, 128)])
                pltpu.sync_copy(stg_d.at[pl.ds(b * 288, 128)],
                                bdst_out.at[pl.ds(seg, 128)])
                full = cur >= 128

                @pl.when(full)
                def _(b=b):
                    for q in range(8):
                        stg_s[pl.ds(b * 288 + q * 16, 16)] = (
                            stg_s[pl.ds(b * 288 + 128 + q * 16, 16)])
                        stg_d[pl.ds(b * 288 + q * 16, 16)] = (
                            stg_d[pl.ds(b * 288 + 128 + q * 16, 16)])
                dec = jnp.where(full, 128, 0)
                if b < 16:
                    nclo = nclo - jnp.where(lane == b, dec, 0)
                else:
                    nchi = nchi - jnp.where(lane == (b - 16), dec, 0)
                outs.append(jnp.where(full, nfl + 1, nfl))
            return (nclo, nchi) + tuple(outs)

        init = ((jnp.zeros((16,), jnp.int32), jnp.zeros((16,), jnp.int32))
                + tuple([jnp.int32(0)] * 32))
        fin = lax.fori_loop(0, NCH, chunk, init)
        fclo, fchi = fin[0], fin[1]

        lv_lo = jnp.zeros((16,), jnp.int32)
        lv_hi = jnp.zeros((16,), jnp.int32)
        for b in range(32):
            cur = (fclo[b] if b < 16 else fchi[b - 16])
            nfl = fin[2 + b]
            for q in range(8):
                stg_s[pl.ds(b * 288 + cur + q * 16, 16)] = (
                    jnp.zeros((16,), jnp.int32))
                stg_d[pl.ds(b * 288 + cur + q * 16, 16)] = (
                    jnp.zeros((16,), jnp.int32) + NPAD)
            seg = (b * 32 + t) * CAPSEG + nfl * 128
            pltpu.sync_copy(stg_s.at[pl.ds(b * 288, 128)],
                            bsrc_out.at[pl.ds(seg, 128)])
            pltpu.sync_copy(stg_d.at[pl.ds(b * 288, 128)],
                            bdst_out.at[pl.ds(seg, 128)])
            tot = jnp.where(cur > 0, nfl + 1, nfl)
            if b < 16:
                lv_lo = jnp.where(lane == b, tot, lv_lo)
            else:
                lv_hi = jnp.where(lane == (b - 16), tot, lv_hi)
        lensb[pl.ds(0, 16)] = lv_lo
        lensb[pl.ds(16, 16)] = lv_hi
        pltpu.sync_copy(lensb, lens_out.at[t])

    return bink


def _make_scA(D):
    mesh = plsc.VectorSubcoreMesh(core_axis_name="c", subcore_axis_name="s")
    K = D // 16

    @functools.partial(
        pl.kernel, mesh=mesh,
        out_type=[jax.ShapeDtypeStruct((EPB,), jnp.float32),
                  jax.ShapeDtypeStruct((32, 16), jnp.float32)],
        scratch_types=[
            pltpu.VMEM((D,), jnp.float32),
            pltpu.VMEM((32, 32), jnp.int32),
            pltpu.VMEM((C,), jnp.int32),
            pltpu.VMEM((C,), jnp.int32),
            pltpu.VMEM((C,), jnp.int32),
            pltpu.VMEM((C, 128), jnp.float32),
            pltpu.VMEM((C, 128), jnp.float32),
            pltpu.VMEM((C,), jnp.float32),
            pltpu.VMEM((16,), jnp.float32),
            pltpu.SemaphoreType.DMA,
            pltpu.SemaphoreType.DMA,
        ])
    def scA(t_hbm, bsrc, bdst, lens_hbm, att_hbm, e_out, tmax_out,
            attv, lenv, sidx, didx, dcl, urows, vrows, ebuf, mbuf,
            sem1, sem2):
        c = lax.axis_index("c")
        s = lax.axis_index("s")
        w = s * 2 + c
        lane = lax.iota(jnp.int32, 16)
        pltpu.sync_copy(att_hbm, attv)
        pltpu.sync_copy(lens_hbm, lenv)

        def segment(tb, mvec):
            nch = _pick32(lenv[tb, pl.ds(0, 16)], lenv[tb, pl.ds(16, 16)], w)
            seg = (w * 32 + tb) * CAPSEG

            def chunk(i, mv):
                off = seg + i * C
                pltpu.sync_copy(bsrc.at[pl.ds(off, C)], sidx)
                pltpu.sync_copy(bdst.at[pl.ds(off, C)], didx)
                for g in range(C // 16):
                    dv = didx[pl.ds(g * 16, 16)]
                    dcl[pl.ds(g * 16, 16)] = jnp.minimum(dv, N - 1)
                cp1 = pltpu.async_copy(t_hbm.at[sidx], urows, sem1)
                cp2 = pltpu.async_copy(t_hbm.at[dcl], vrows, sem2)
                cp1.wait()
                cp2.wait()

                def group(g, mvv):
                    evec = jnp.zeros((16,), jnp.float32)
                    for j16 in range(16):
                        j = g * 16 + j16
                        acc = jnp.zeros((16,), jnp.float32)
                        for k in range(K):
                            tv = (urows[j, pl.ds(k * 16, 16)]
                                  + vrows[j, pl.ds(64 + k * 16, 16)])
                            lv = jnp.maximum(tv, 0.2 * tv)
                            acc = acc + lv * attv[pl.ds(k * 16, 16)]
                        evec = jnp.where(lane == j16, _vsum16(acc), evec)
                    ebuf[pl.ds(g * 16, 16)] = evec
                    return jnp.maximum(mvv, evec)
                mv = lax.fori_loop(0, C // 16, group, mv)
                pltpu.sync_copy(ebuf, e_out.at[pl.ds(off, C)])
                return mv

            return lax.fori_loop(0, nch, chunk, mvec)

        mvec = lax.fori_loop(0, 32, segment,
                             jnp.full((16,), NEG, jnp.float32))
        mbuf[...] = mvec
        pltpu.sync_copy(mbuf, tmax_out.at[w])

    return scA


def _make_scB(D):
    mesh = plsc.VectorSubcoreMesh(core_axis_name="c", subcore_axis_name="s")
    K = D // 16
    NR = OWN // 16

    @functools.partial(
        pl.kernel, mesh=mesh,
        out_type=jax.ShapeDtypeStruct((NPAD, D), jnp.float32),
        scratch_types=[
            pltpu.VMEM((32, 16), jnp.float32),
            pltpu.VMEM((D,), jnp.float32),
            pltpu.VMEM((32, 32), jnp.int32),
            pltpu.VMEM((C,), jnp.int32),
            pltpu.VMEM((C,), jnp.int32),
            pltpu.VMEM((C, 128), jnp.float32),
            pltpu.VMEM((C,), jnp.float32),
            pltpu.VMEM(((OWN + 1) * D,), jnp.float32),
            pltpu.VMEM(((NR + 1) * 16,), jnp.float32),
            pltpu.VMEM((16, D), jnp.float32),
            pltpu.SemaphoreType.DMA,
        ])
    def scB(t_hbm, e_hbm, bsrc, bdst, lens_hbm, tmax_hbm, bias_hbm,
            raw_out, tmv, biasv, lenv, sidx, didx, urows, exb,
            acc, den, outb, sem1):
        c = lax.axis_index("c")
        s = lax.axis_index("s")
        w = s * 2 + c
        lane = lax.iota(jnp.int32, 16)
        pltpu.sync_copy(tmax_hbm, tmv)
        pltpu.sync_copy(bias_hbm, biasv)
        pltpu.sync_copy(lens_hbm, lenv)
        gmax = _vmax16(tmv[w])

        def zrow(j, carry):
            acc[pl.ds(j * 16, 16)] = jnp.zeros((16,), jnp.float32)
            return carry
        lax.fori_loop(0, (OWN + 1) * D // 16, zrow, 0)

        def zden(j, carry):
            den[pl.ds(j * 16, 16)] = jnp.zeros((16,), jnp.float32)
            return carry
        lax.fori_loop(0, NR + 1, zden, 0)

        nbase = w * OWN

        def segment(tb, scarry):
            nch = _pick32(lenv[tb, pl.ds(0, 16)], lenv[tb, pl.ds(16, 16)], w)
            seg = (w * 32 + tb) * CAPSEG

            def chunk(i, carry):
                off = seg + i * C
                pltpu.sync_copy(bsrc.at[pl.ds(off, C)], sidx)
                cp = pltpu.async_copy(t_hbm.at[sidx], urows, sem1)
                pltpu.sync_copy(bdst.at[pl.ds(off, C)], didx)
                pltpu.sync_copy(e_hbm.at[pl.ds(off, C)], exb)
                for g in range(C // 16):
                    ev = exb[pl.ds(g * 16, 16)]
                    exb[pl.ds(g * 16, 16)] = jnp.exp(ev - gmax)
                cp.wait()

                def egroup(g, cc):
                    exv = exb[pl.ds(g * 16, 16)]
                    dv = didx[pl.ds(g * 16, 16)] - nbase
                    ok = (dv >= 0) & (dv < OWN)
                    dl = jnp.where(ok, dv, DUMPT)
                    for j16 in range(16):
                        j = g * 16 + j16
                        sc = exv[j16]
                        row = dl[j16]
                        rb = row * D
                        for k in range(K):
                            acc[pl.ds(rb + k * 16, 16)] = (
                                acc[pl.ds(rb + k * 16, 16)]
                                + urows[j, pl.ds(k * 16, 16)] * sc)
                        dbase = jnp.left_shift(jnp.right_shift(row, 4), 4)
                        dcol = row & 15
                        den[pl.ds(dbase, 16)] = (
                            den[pl.ds(dbase, 16)]
                            + jnp.where(lane == dcol, sc, 0.0))
                    return cc
                lax.fori_loop(0, C // 16, egroup, 0)
                return carry

            lax.fori_loop(0, nch, chunk, 0)
            return scarry

        lax.fori_loop(0, 32, segment, 0)

        def fin(rg, carry):
            rv = 1.0 / (den[pl.ds(rg * 16, 16)] + 1e-16)
            for j16 in range(16):
                row = rg * 16 + j16
                rcp = rv[j16]
                for k in range(K):
                    outb[j16, pl.ds(k * 16, 16)] = (
                        acc[pl.ds(row * D + k * 16, 16)] * rcp
                        + biasv[pl.ds(k * 16, 16)])
            pltpu.sync_copy(outb, raw_out.at[pl.ds(nbase + rg * 16, 16)])
            return carry
        lax.fori_loop(0, NR, fin, 0)

    return scB


def kernel(x, edge_index, params):
    src = edge_index[0].astype(jnp.int32)
    dst = edge_index[1].astype(jnp.int32)
    src_p = jnp.concatenate([src, jnp.zeros((EP - E,), jnp.int32)])
    dst_p = jnp.concatenate([dst, jnp.full((EP - E,), NPAD, jnp.int32)])

    bink = _make_bin()
    bsrc, bdst, lens = bink(src_p, dst_p)
    if _CBISECT == 1:
        return (jnp.zeros((N, 10), jnp.float32) + bsrc[0] + bdst[0]
                + lens[0, 0])

    scA64, scB64 = _make_scA(64), _make_scB(64)
    scA16, scB16 = _make_scA(16), _make_scB(16)

    raw = None
    for i in range(4):
        p = params[f'layer{i}']
        wl, wr, att, b = p['Wl'], p['Wr'], p['att'], p['b']
        if i == 3:
            wl = jnp.pad(wl, ((0, 0), (0, 6)))
            wr = jnp.pad(wr, ((0, 0), (0, 6)))
            att = jnp.pad(att, (0, 6))
            b = jnp.pad(b, (0, 6))
        if i == 0:
            t = _tc0(x, wl, wr)
        else:
            nrm = params[f'norm{i-1}']
            t = _tcmid(raw[:N], nrm['g'].reshape(1, -1),
                       nrm['b'].reshape(1, -1), wl, wr)
        scA = scA64 if wl.shape[1] == 64 else scA16
        scB = scB64 if wl.shape[1] == 64 else scB16
        e, tmax = scA(t, bsrc, bdst, lens, att)
        if _CBISECT == 2:
            return jnp.zeros((N, 10), jnp.float32) + e[0] + tmax[0, 0]
        raw = scB(t, e, bsrc, bdst, lens, tmax, b)
    return raw[:N, :10]


_CBISECT = 0


# speedup vs baseline: 2.4907x; 1.0001x over previous
import functools

import jax
import jax.numpy as jnp
from jax import lax
from jax.experimental import pallas as pl
from jax.experimental.pallas import tpu as pltpu
from jax.experimental.pallas import tpu_sc as plsc

N = 50000
E = 800000
OWN = 1568
NPAD = 32 * OWN
DUMPT = OWN
EP = 802816
TPA = EP // 32
C = 128
CAPSEG = TPA + 128
NSEG = 32 * 32
EPB = NSEG * CAPSEG
NEG = -3.0e38


def _tc0_body(x_ref, wl_ref, wr_ref, t_ref):
    xb = x_ref[...]
    dout = wl_ref.shape[1]
    if dout < 64:
        t_ref[...] = jnp.zeros_like(t_ref)
    t_ref[:, 0:dout] = jnp.dot(xb, wl_ref[...],
                               preferred_element_type=jnp.float32)
    t_ref[:, 64:64 + dout] = jnp.dot(xb, wr_ref[...],
                                     preferred_element_type=jnp.float32)


def _tc0(x, wl, wr):
    BN = 2000
    dout = wl.shape[1]
    return pl.pallas_call(
        _tc0_body,
        grid=(N // BN,),
        in_specs=[pl.BlockSpec((BN, 64), lambda i: (i, 0)),
                  pl.BlockSpec((64, dout), lambda i: (0, 0)),
                  pl.BlockSpec((64, dout), lambda i: (0, 0))],
        out_specs=[pl.BlockSpec((BN, 128), lambda i: (i, 0))],
        out_shape=[jax.ShapeDtypeStruct((N, 128), jnp.float32)],
    )(x, wl, wr)[0]


def _tcmid_body(r_ref, g_ref, b_ref, wl_ref, wr_ref, t_ref):
    t = r_ref[...]
    mu = jnp.mean(t, axis=1, keepdims=True)
    d = t - mu
    var = jnp.mean(d * d, axis=1, keepdims=True)
    y = d * lax.rsqrt(var + 1e-5) * g_ref[...] + b_ref[...]
    h = y * jax.nn.sigmoid(y)
    dout = wl_ref.shape[1]
    if dout < 64:
        t_ref[...] = jnp.zeros_like(t_ref)
    t_ref[:, 0:dout] = jnp.dot(h, wl_ref[...],
                               preferred_element_type=jnp.float32)
    t_ref[:, 64:64 + dout] = jnp.dot(h, wr_ref[...],
                                     preferred_element_type=jnp.float32)


def _tcmid(raw, g, b, wl, wr):
    BN = 2000
    dout = wl.shape[1]
    return pl.pallas_call(
        _tcmid_body,
        grid=(N // BN,),
        in_specs=[pl.BlockSpec((BN, 64), lambda i: (i, 0)),
                  pl.BlockSpec((1, 64), lambda i: (0, 0)),
                  pl.BlockSpec((1, 64), lambda i: (0, 0)),
                  pl.BlockSpec((64, dout), lambda i: (0, 0)),
                  pl.BlockSpec((64, dout), lambda i: (0, 0))],
        out_specs=[pl.BlockSpec((BN, 128), lambda i: (i, 0))],
        out_shape=[jax.ShapeDtypeStruct((N, 128), jnp.float32)],
    )(raw, g, b, wl, wr)[0]


def _vsum16(x):
    lane = lax.iota(jnp.int32, 16)
    for sh in (8, 4, 2, 1):
        x = x + x.at[lane ^ sh].get(mode="promise_in_bounds")
    return x


def _vmax16(x):
    lane = lax.iota(jnp.int32, 16)
    for sh in (8, 4, 2, 1):
        x = jnp.maximum(x, x.at[lane ^ sh].get(mode="promise_in_bounds"))
    return x


def _pick32(lo, hi, w):
    lane = lax.iota(jnp.int32, 16)
    a = jnp.where(lane == w, lo, jnp.zeros_like(lo))
    b = jnp.where(lane == (w - 16), hi, jnp.zeros_like(hi))
    return _vsum16(a + b)[0]


def _make_bin():
    mesh = plsc.VectorSubcoreMesh(core_axis_name="c", subcore_axis_name="s")
    NCH = TPA // C

    @functools.partial(
        pl.kernel, mesh=mesh,
        out_type=[jax.ShapeDtypeStruct((EPB,), jnp.int32),
                  jax.ShapeDtypeStruct((EPB,), jnp.int32),
                  jax.ShapeDtypeStruct((32, 32), jnp.int32)],
        scratch_types=[
            pltpu.VMEM((C,), jnp.int32),
            pltpu.VMEM((C,), jnp.int32),
            pltpu.VMEM((9216,), jnp.int32),
            pltpu.VMEM((9216,), jnp.int32),
            pltpu.VMEM((32,), jnp.int32),
        ])
    def bink(src_hbm, dst_hbm, bsrc_out, bdst_out, lens_out,
             sidx, didx, stg_s, stg_d, lensb):
        c = lax.axis_index("c")
        s = lax.axis_index("s")
        t = s * 2 + c
        base = t * TPA
        lane = lax.iota(jnp.int32, 16)

        def chunk(i, carry):
            off = base + i * C
            pltpu.sync_copy(src_hbm.at[pl.ds(off, C)], sidx)
            pltpu.sync_copy(dst_hbm.at[pl.ds(off, C)], didx)
            clo, chi = carry[0], carry[1]
            nfls = list(carry[2:])
            for g in range(C // 16):
                sv = sidx[pl.ds(g * 16, 16)]
                dv = didx[pl.ds(g * 16, 16)]
                bv = jnp.minimum(
                    jnp.right_shift(
                        jnp.right_shift(dv, 5) * 1338, 16), 31)
                rank = jnp.zeros((16,), jnp.int32)
                for sh in range(1, 16):
                    idxc = jnp.maximum(lane - sh, 0)
                    gsh = bv.at[idxc].get(mode="promise_in_bounds")
                    rank = rank + jnp.where((lane >= sh) & (gsh == bv),
                                            1, 0)
                cl = clo.at[jnp.minimum(bv, 15)].get(
                    mode="promise_in_bounds")
                ch = chi.at[jnp.maximum(bv - 16, 0)].get(
                    mode="promise_in_bounds")
                dest = bv * 288 + jnp.where(bv < 16, cl, ch) + rank
                for j16 in range(16):
                    p = dest[j16]
                    stg_s[pl.ds(p, 16)] = (jnp.zeros((16,), jnp.int32)
                                           + sv[j16])
                    stg_d[pl.ds(p, 16)] = (jnp.zeros((16,), jnp.int32)
                                           + dv[j16])
                    bj = bv[j16]
                    clo = clo + jnp.where(lane == bj, 1, 0)
                    chi = chi + jnp.where(lane == (bj - 16), 1, 0)
            outs = []
            nclo, nchi = clo, chi
            for b in range(32):
                cur = (clo[b] if b < 16 else chi[b - 16])
                nfl = nfls[b]
                seg = (b * 32 + t) * CAPSEG + nfl * 128
                pltpu.sync_copy(stg_s.at[pl.ds(b * 288, 128)],
                                bsrc_out.at[pl.ds(seg, 128)])
                pltpu.sync_copy(stg_d.at[pl.ds(b * 288, 128)],
                                bdst_out.at[pl.ds(seg, 128)])
                full = cur >= 128

                @pl.when(full)
                def _(b=b):
                    for q in range(8):
                        stg_s[pl.ds(b * 288 + q * 16, 16)] = (
                            stg_s[pl.ds(b * 288 + 128 + q * 16, 16)])
                        stg_d[pl.ds(b * 288 + q * 16, 16)] = (
                            stg_d[pl.ds(b * 288 + 128 + q * 16, 16)])
                dec = jnp.where(full, 128, 0)
                if b < 16:
                    nclo = nclo - jnp.where(lane == b, dec, 0)
                else:
                    nchi = nchi - jnp.where(lane == (b - 16), dec, 0)
                outs.append(jnp.where(full, nfl + 1, nfl))
            return (nclo, nchi) + tuple(outs)

        init = ((jnp.zeros((16,), jnp.int32), jnp.zeros((16,), jnp.int32))
                + tuple([jnp.int32(0)] * 32))
        fin = lax.fori_loop(0, NCH, chunk, init)
        fclo, fchi = fin[0], fin[1]

        lv_lo = jnp.zeros((16,), jnp.int32)
        lv_hi = jnp.zeros((16,), jnp.int32)
        for b in range(32):
            cur = (fclo[b] if b < 16 else fchi[b - 16])
            nfl = fin[2 + b]
            for q in range(8):
                stg_s[pl.ds(b * 288 + cur + q * 16, 16)] = (
                    jnp.zeros((16,), jnp.int32))
                stg_d[pl.ds(b * 288 + cur + q * 16, 16)] = (
                    jnp.zeros((16,), jnp.int32) + NPAD)
            seg = (b * 32 + t) * CAPSEG + nfl * 128
            pltpu.sync_copy(stg_s.at[pl.ds(b * 288, 128)],
                            bsrc_out.at[pl.ds(seg, 128)])
            pltpu.sync_copy(stg_d.at[pl.ds(b * 288, 128)],
                            bdst_out.at[pl.ds(seg, 128)])
            tot = jnp.where(cur > 0, nfl + 1, nfl)
            if b < 16:
                lv_lo = jnp.where(lane == b, tot, lv_lo)
            else:
                lv_hi = jnp.where(lane == (b - 16), tot, lv_hi)
        lensb[pl.ds(0, 16)] = lv_lo
        lensb[pl.ds(16, 16)] = lv_hi
        pltpu.sync_copy(lensb, lens_out.at[t])

    return bink


def _make_scA(D):
    mesh = plsc.VectorSubcoreMesh(core_axis_name="c", subcore_axis_name="s")
    K = D // 16

    @functools.partial(
        pl.kernel, mesh=mesh,
        out_type=[jax.ShapeDtypeStruct((EPB,), jnp.float32),
                  jax.ShapeDtypeStruct((32, 16), jnp.float32)],
        scratch_types=[
            pltpu.VMEM((D,), jnp.float32),
            pltpu.VMEM((32, 32), jnp.int32),
            pltpu.VMEM((C,), jnp.int32),
            pltpu.VMEM((C,), jnp.int32),
            pltpu.VMEM((C,), jnp.int32),
            pltpu.VMEM((C, 128), jnp.float32),
            pltpu.VMEM((C, 128), jnp.float32),
            pltpu.VMEM((C,), jnp.float32),
            pltpu.VMEM((16,), jnp.float32),
            pltpu.SemaphoreType.DMA,
            pltpu.SemaphoreType.DMA,
        ])
    def scA(t_hbm, bsrc, bdst, lens_hbm, att_hbm, e_out, tmax_out,
            attv, lenv, sidx, didx, dcl, urows, vrows, ebuf, mbuf,
            sem1, sem2):
        c = lax.axis_index("c")
        s = lax.axis_index("s")
        w = s * 2 + c
        lane = lax.iota(jnp.int32, 16)
        pltpu.sync_copy(att_hbm, attv)
        pltpu.sync_copy(lens_hbm, lenv)

        def segment(tb, mvec):
            nch = _pick32(lenv[tb, pl.ds(0, 16)], lenv[tb, pl.ds(16, 16)], w)
            seg = (w * 32 + tb) * CAPSEG

            def chunk(i, mv):
                off = seg + i * C
                pltpu.sync_copy(bsrc.at[pl.ds(off, C)], sidx)
                pltpu.sync_copy(bdst.at[pl.ds(off, C)], didx)
                for g in range(C // 16):
                    dv = didx[pl.ds(g * 16, 16)]
                    dcl[pl.ds(g * 16, 16)] = jnp.minimum(dv, N - 1)
                cp1 = pltpu.async_copy(t_hbm.at[sidx], urows, sem1)
                cp2 = pltpu.async_copy(t_hbm.at[dcl], vrows, sem2)
                cp1.wait()
                cp2.wait()

                def group(g, mvv):
                    evec = jnp.zeros((16,), jnp.float32)
                    for j16 in range(16):
                        j = g * 16 + j16
                        acc = jnp.zeros((16,), jnp.float32)
                        for k in range(K):
                            tv = (urows[j, pl.ds(k * 16, 16)]
                                  + vrows[j, pl.ds(64 + k * 16, 16)])
                            lv = jnp.maximum(tv, 0.2 * tv)
                            acc = acc + lv * attv[pl.ds(k * 16, 16)]
                        evec = jnp.where(lane == j16, _vsum16(acc), evec)
                    ebuf[pl.ds(g * 16, 16)] = evec
                    return jnp.maximum(mvv, evec)
                mv = lax.fori_loop(0, C // 16, group, mv)
                pltpu.sync_copy(ebuf, e_out.at[pl.ds(off, C)])
                return mv

            return lax.fori_loop(0, nch, chunk, mvec)

        mvec = lax.fori_loop(0, 32, segment,
                             jnp.full((16,), NEG, jnp.float32))
        mbuf[...] = mvec
        pltpu.sync_copy(mbuf, tmax_out.at[w])

    return scA


def _make_scB(D):
    mesh = plsc.VectorSubcoreMesh(core_axis_name="c", subcore_axis_name="s")
    K = D // 16
    NR = OWN // 16

    @functools.partial(
        pl.kernel, mesh=mesh,
        out_type=jax.ShapeDtypeStruct((NPAD, D), jnp.float32),
        scratch_types=[
            pltpu.VMEM((32, 16), jnp.float32),
            pltpu.VMEM((D,), jnp.float32),
            pltpu.VMEM((32, 32), jnp.int32),
            pltpu.VMEM((C,), jnp.int32),
            pltpu.VMEM((C,), jnp.int32),
            pltpu.VMEM((C, 128), jnp.float32),
            pltpu.VMEM((C,), jnp.float32),
            pltpu.VMEM(((OWN + 1) * D,), jnp.float32),
            pltpu.VMEM(((NR + 1) * 16,), jnp.float32),
            pltpu.VMEM((16, D), jnp.float32),
            pltpu.SemaphoreType.DMA,
        ])
    def scB(t_hbm, e_hbm, bsrc, bdst, lens_hbm, tmax_hbm, bias_hbm,
            raw_out, tmv, biasv, lenv, sidx, didx, urows, exb,
            acc, den, outb, sem1):
        c = lax.axis_index("c")
        s = lax.axis_index("s")
        w = s * 2 + c
        lane = lax.iota(jnp.int32, 16)
        pltpu.sync_copy(tmax_hbm, tmv)
        pltpu.sync_copy(bias_hbm, biasv)
        pltpu.sync_copy(lens_hbm, lenv)
        gmax = _vmax16(tmv[w])

        def zrow(j, carry):
            acc[pl.ds(j * 16, 16)] = jnp.zeros((16,), jnp.float32)
            return carry
        lax.fori_loop(0, (OWN + 1) * D // 16, zrow, 0)

        def zden(j, carry):
            den[pl.ds(j * 16, 16)] = jnp.zeros((16,), jnp.float32)
            return carry
        lax.fori_loop(0, NR + 1, zden, 0)

        nbase = w * OWN

        def segment(tb, scarry):
            nch = _pick32(lenv[tb, pl.ds(0, 16)], lenv[tb, pl.ds(16, 16)], w)
            seg = (w * 32 + tb) * CAPSEG

            def chunk(i, carry):
                off = seg + i * C
                pltpu.sync_copy(bsrc.at[pl.ds(off, C)], sidx)
                cp = pltpu.async_copy(t_hbm.at[sidx], urows, sem1)
                pltpu.sync_copy(bdst.at[pl.ds(off, C)], didx)
                pltpu.sync_copy(e_hbm.at[pl.ds(off, C)], exb)
                for g in range(C // 16):
                    ev = exb[pl.ds(g * 16, 16)]
                    exb[pl.ds(g * 16, 16)] = jnp.exp(ev - gmax)
                cp.wait()

                def egroup(g, cc):
                    exv = exb[pl.ds(g * 16, 16)]
                    dv = didx[pl.ds(g * 16, 16)] - nbase
                    ok = (dv >= 0) & (dv < OWN)
                    dl = jnp.where(ok, dv, DUMPT)
                    for j16 in range(16):
                        j = g * 16 + j16
                        sc = exv[j16]
                        row = dl[j16]
                        rb = row * D
                        for k in range(K):
                            acc[pl.ds(rb + k * 16, 16)] = (
                                acc[pl.ds(rb + k * 16, 16)]
                                + urows[j, pl.ds(k * 16, 16)] * sc)
                        dbase = jnp.left_shift(jnp.right_shift(row, 4), 4)
                        dcol = row & 15
                        den[pl.ds(dbase, 16)] = (
                            den[pl.ds(dbase, 16)]
                            + jnp.where(lane == dcol, sc, 0.0))
                    return cc
                lax.fori_loop(0, C // 16, egroup, 0)
                return carry

            lax.fori_loop(0, nch, chunk, 0)
            return scarry

        lax.fori_loop(0, 32, segment, 0)

        def fin(rg, carry):
            rv = 1.0 / (den[pl.ds(rg * 16, 16)] + 1e-16)
            for j16 in range(16):
                row = rg * 16 + j16
                rcp = rv[j16]
                for k in range(K):
                    outb[j16, pl.ds(k * 16, 16)] = (
                        acc[pl.ds(row * D + k * 16, 16)] * rcp
                        + biasv[pl.ds(k * 16, 16)])
            pltpu.sync_copy(outb, raw_out.at[pl.ds(nbase + rg * 16, 16)])
            return carry
        lax.fori_loop(0, NR, fin, 0)

    return scB


def kernel(x, edge_index, params):
    src = edge_index[0].astype(jnp.int32)
    dst = edge_index[1].astype(jnp.int32)
    src_p = jnp.concatenate([src, jnp.zeros((EP - E,), jnp.int32)])
    dst_p = jnp.concatenate([dst, jnp.full((EP - E,), NPAD, jnp.int32)])

    bink = _make_bin()
    bsrc, bdst, lens = bink(src_p, dst_p)

    scA64, scB64 = _make_scA(64), _make_scB(64)
    scA16, scB16 = _make_scA(16), _make_scB(16)

    raw = None
    for i in range(4):
        p = params[f'layer{i}']
        wl, wr, att, b = p['Wl'], p['Wr'], p['att'], p['b']
        if i == 3:
            wl = jnp.pad(wl, ((0, 0), (0, 6)))
            wr = jnp.pad(wr, ((0, 0), (0, 6)))
            att = jnp.pad(att, (0, 6))
            b = jnp.pad(b, (0, 6))
        if i == 0:
            t = _tc0(x, wl, wr)
        else:
            nrm = params[f'norm{i-1}']
            t = _tcmid(raw[:N], nrm['g'].reshape(1, -1),
                       nrm['b'].reshape(1, -1), wl, wr)
        scA = scA64 if wl.shape[1] == 64 else scA16
        scB = scB64 if wl.shape[1] == 64 else scB16
        e, tmax = scA(t, bsrc, bdst, lens, att)
        raw = scB(t, e, bsrc, bdst, lens, tmax, b)
    return raw[:N, :10]
